# baseline (speedup 1.0000x reference)
import functools

import jax
import jax.numpy as jnp
from jax import lax
from jax.experimental import pallas as pl
from jax.experimental.pallas import tpu as pltpu
from jax.experimental.pallas import tpu_sc as plsc

N = 10000
D = 128
FOUT = 2
NPAD = 10112
DUMMY = N
NC, NS = 2, 16
NW = NC * NS
K = 120
BLK = 8
NBLK = 11
CPT = NBLK * BLK
EPAD = NW * CPT * K
RPT = NPAD // NS

_mesh = plsc.VectorSubcoreMesh(core_axis_name="c", subcore_axis_name="s",
                               num_cores=NC, num_subcores=NS)


@functools.partial(
    pl.kernel,
    mesh=_mesh,
    out_type=jax.ShapeDtypeStruct((NC, NPAD, 16), jnp.float32),
    scratch_types=[
        pltpu.VMEM((CPT, K), jnp.int32),
        pltpu.VMEM((K, 16), jnp.float32),
        pltpu.VMEM_SHARED((NPAD, 16), jnp.float32),
    ],
)
def _degree_histogram(dst_hbm, ones_hbm, zeros_hbm, out_hbm,
                      idx_v, ones_v, acc_sh):
    c = lax.axis_index("c")
    s = lax.axis_index("s")
    w = c * NS + s
    pltpu.sync_copy(dst_hbm.at[w], idx_v)
    pltpu.sync_copy(ones_hbm, ones_v)
    r0 = s * RPT
    pltpu.sync_copy(zeros_hbm.at[pl.ds(r0, RPT)], acc_sh.at[pl.ds(r0, RPT)])
    plsc.subcore_barrier()

    @pl.loop(0, CPT)
    def _(j):
        pltpu.sync_copy(ones_v, acc_sh.at[idx_v.at[j]], add=True)

    plsc.subcore_barrier()
    pltpu.sync_copy(acc_sh.at[pl.ds(r0, RPT)], out_hbm.at[c, pl.ds(r0, RPT)])


@functools.partial(
    pl.kernel,
    mesh=_mesh,
    out_type=jax.ShapeDtypeStruct((NC, NPAD, D), jnp.float32),
    scratch_types=[
        pltpu.VMEM((CPT, K), jnp.int32),
        pltpu.VMEM((BLK, K), jnp.int32),
        pltpu.VMEM((BLK, K), jnp.int32),
        pltpu.VMEM((K, D), jnp.float32),
        pltpu.VMEM((K, D), jnp.float32),
        pltpu.SemaphoreType.DMA,
        pltpu.SemaphoreType.DMA,
        pltpu.SemaphoreType.DMA,
        pltpu.SemaphoreType.DMA,
        pltpu.VMEM_SHARED((NPAD, D), jnp.float32),
    ],
)
def _segment_scatter(h2_hbm, src_hbm, dst_hbm, zeros_hbm, out_hbm,
                     src_v, db0, db1, rows0, rows1,
                     g0, g1, i0, i1, acc_sh):
    dblk, rows, gsem, dsem = (db0, db1), (rows0, rows1), (g0, g1), (i0, i1)
    c = lax.axis_index("c")
    s = lax.axis_index("s")
    w = c * NS + s

    def dst_start(t, p):
        pltpu.async_copy(dst_hbm.at[w, t], dblk[p], dsem[p])

    def dst_wait(p):
        pltpu.make_async_copy(dst_hbm.at[w, 0], dblk[p], dsem[p]).wait()

    pltpu.sync_copy(src_hbm.at[w], src_v)
    dst_start(0, 0)
    r0 = s * RPT
    pltpu.sync_copy(zeros_hbm.at[pl.ds(r0, RPT)], acc_sh.at[pl.ds(r0, RPT)])
    plsc.subcore_barrier()

    @pl.loop(0, NBLK // 2)
    def _(g):
        t0 = 2 * g
        for l in range(2 * BLK):
            p = l % 2
            bp = (l // BLK) % 2
            if l == 0:
                dst_wait(0)
                dst_start(t0 + 1, 1)
            if l == BLK:
                dst_wait(1)
                dst_start(t0 + 2, 0)
            pltpu.async_copy(h2_hbm.at[src_v.at[16 * g + l]], rows[p],
                             gsem[p]).wait()
            pltpu.sync_copy(rows[p], acc_sh.at[dblk[bp].at[l % BLK]],
                            add=True)

    dst_wait(0)
    for l in range(BLK):
        p = l % 2
        pltpu.async_copy(h2_hbm.at[src_v.at[CPT - BLK + l]], rows[p],
                         gsem[p]).wait()
        pltpu.sync_copy(rows[p], acc_sh.at[dblk[0].at[l]], add=True)

    plsc.subcore_barrier()
    pltpu.sync_copy(acc_sh.at[pl.ds(r0, RPT)], out_hbm.at[c, pl.ds(r0, RPT)])


def _matmul_body(x_ref, w1_ref, h_ref):
    h_ref[...] = jnp.dot(x_ref[...], w1_ref[...],
                         preferred_element_type=jnp.float32)


def _scale_body(h_ref, degacc_ref, h2_ref):
    deg = degacc_ref[0, :, 0:1] + degacc_ref[1, :, 0:1]
    dinv = lax.rsqrt(deg[:N])
    h2_ref[...] = h_ref[...] * dinv


def _combine_body(s_ref, degacc_ref, b1_ref, w2_ref, b2_ref, out_ref):
    deg = degacc_ref[0, :, 0:1] + degacc_ref[1, :, 0:1]
    dinv = lax.rsqrt(deg[:N])
    srows = s_ref[0, :N, :] + s_ref[1, :N, :]
    rows = jnp.maximum(srows * dinv + b1_ref[...], 0.0)
    g = jnp.sum(rows, axis=0, keepdims=True) * (1.0 / N)
    logits = jnp.dot(g, w2_ref[...], preferred_element_type=jnp.float32)
    logits = logits + b2_ref[...]
    m = jnp.max(logits, axis=1, keepdims=True)
    e = jnp.exp(logits - m)
    out_ref[...] = e / jnp.sum(e, axis=1, keepdims=True)


def kernel(x, edge_index, W1, b1, W2, b2):
    e = edge_index.shape[1]
    iota = jnp.arange(N, dtype=jnp.int32)
    npad_e = EPAD - (e + N)
    src_all = jnp.concatenate(
        [edge_index[0], iota, jnp.zeros((npad_e,), jnp.int32)])
    pad_dst = DUMMY + jnp.arange(npad_e, dtype=jnp.int32) % (NPAD - N)
    dst_all = jnp.concatenate([edge_index[1], iota, pad_dst])
    src3 = src_all.reshape(NW, CPT, K)
    dst4 = dst_all.reshape(NW, NBLK, BLK, K)
    dst3 = dst_all.reshape(NW, CPT, K)
    ones16 = jnp.ones((K, 16), jnp.float32)
    zeros16 = jnp.zeros((NPAD, 16), jnp.float32)
    zeros_d = jnp.zeros((NPAD, D), jnp.float32)

    degacc = _degree_histogram(dst3, ones16, zeros16)

    h = pl.pallas_call(
        _matmul_body,
        out_shape=jax.ShapeDtypeStruct((N, D), jnp.float32),
    )(x, W1)

    h2 = pl.pallas_call(
        _scale_body,
        out_shape=jax.ShapeDtypeStruct((N, D), jnp.float32),
    )(h, degacc)

    seg = _segment_scatter(h2, src3, dst4, zeros_d)

    out = pl.pallas_call(
        _combine_body,
        out_shape=jax.ShapeDtypeStruct((1, FOUT), jnp.float32),
    )(seg, degacc, b1.reshape(1, D), W2, b2.reshape(1, FOUT))
    return out


# speedup vs baseline: 1.8912x; 1.8912x over previous
import functools

import jax
import jax.numpy as jnp
from jax import lax
from jax.experimental import pallas as pl
from jax.experimental.pallas import tpu as pltpu
from jax.experimental.pallas import tpu_sc as plsc

N = 10000
D = 128
FOUT = 2
NPAD = 10112
DUMMY = N
NC, NS = 2, 16
NW = NC * NS
K = 128
CPT = 81
EPAD = NW * CPT * K
RPT = NPAD // NS

_mesh = plsc.VectorSubcoreMesh(core_axis_name="c", subcore_axis_name="s",
                               num_cores=NC, num_subcores=NS)


@functools.partial(
    pl.kernel,
    mesh=_mesh,
    out_type=jax.ShapeDtypeStruct((NC, NPAD, 16), jnp.float32),
    scratch_types=[
        pltpu.VMEM((CPT, K), jnp.int32),
        pltpu.VMEM((K, 16), jnp.float32),
        pltpu.VMEM_SHARED((NPAD, 16), jnp.float32),
    ],
)
def _degree_histogram(dst_hbm, ones_hbm, zeros_hbm, out_hbm,
                      idx_v, ones_v, acc_sh):
    c = lax.axis_index("c")
    s = lax.axis_index("s")
    w = c * NS + s
    pltpu.sync_copy(dst_hbm.at[w], idx_v)
    pltpu.sync_copy(ones_hbm, ones_v)
    r0 = s * RPT
    pltpu.sync_copy(zeros_hbm.at[pl.ds(r0, RPT)], acc_sh.at[pl.ds(r0, RPT)])
    plsc.subcore_barrier()

    @pl.loop(0, CPT)
    def _(j):
        pltpu.sync_copy(ones_v, acc_sh.at[idx_v.at[j]], add=True)

    plsc.subcore_barrier()
    pltpu.sync_copy(acc_sh.at[pl.ds(r0, RPT)], out_hbm.at[c, pl.ds(r0, RPT)])


@functools.partial(
    pl.kernel,
    mesh=_mesh,
    out_type=jax.ShapeDtypeStruct((NC, NPAD, D), jnp.float32),
    scratch_types=[
        pltpu.VMEM((CPT, K), jnp.int32),
        pltpu.VMEM((CPT, K), jnp.int32),
        pltpu.VMEM((K, D), jnp.float32),
        pltpu.VMEM_SHARED((NPAD, D), jnp.float32),
        pltpu.SemaphoreType.DMA,
    ],
)
def _segment_scatter(h2_hbm, src_hbm, dst_hbm, zeros_hbm, out_hbm,
                     src_v, dst_v, rows_v, acc_sh, sem):
    c = lax.axis_index("c")
    s = lax.axis_index("s")
    w = c * NS + s
    pltpu.sync_copy(src_hbm.at[w], src_v)
    pltpu.sync_copy(dst_hbm.at[w], dst_v)
    r0 = s * RPT
    pltpu.sync_copy(zeros_hbm.at[pl.ds(r0, RPT)], acc_sh.at[pl.ds(r0, RPT)])
    plsc.subcore_barrier()

    @pl.loop(0, CPT)
    def _(j):
        pltpu.async_copy(h2_hbm.at[src_v.at[j]], rows_v, sem).wait()
        pltpu.sync_copy(rows_v, acc_sh.at[dst_v.at[j]], add=True)

    plsc.subcore_barrier()
    pltpu.sync_copy(acc_sh.at[pl.ds(r0, RPT)], out_hbm.at[c, pl.ds(r0, RPT)])


def _matmul_body(x_ref, w1_ref, h_ref):
    h_ref[...] = jnp.dot(x_ref[...], w1_ref[...],
                         preferred_element_type=jnp.float32)


def _scale_body(h_ref, degacc_ref, h2_ref):
    deg = degacc_ref[0, :, 0:1] + degacc_ref[1, :, 0:1]
    dinv = lax.rsqrt(deg[:N])
    h2_ref[...] = h_ref[...] * dinv


def _combine_body(s_ref, degacc_ref, b1_ref, w2_ref, b2_ref, out_ref):
    deg = degacc_ref[0, :, 0:1] + degacc_ref[1, :, 0:1]
    dinv = lax.rsqrt(deg[:N])
    srows = s_ref[0, :N, :] + s_ref[1, :N, :]
    rows = jnp.maximum(srows * dinv + b1_ref[...], 0.0)
    g = jnp.sum(rows, axis=0, keepdims=True) * (1.0 / N)
    logits = jnp.dot(g, w2_ref[...], preferred_element_type=jnp.float32)
    logits = logits + b2_ref[...]
    m = jnp.max(logits, axis=1, keepdims=True)
    e = jnp.exp(logits - m)
    out_ref[...] = e / jnp.sum(e, axis=1, keepdims=True)


def kernel(x, edge_index, W1, b1, W2, b2):
    e = edge_index.shape[1]
    iota = jnp.arange(N, dtype=jnp.int32)
    npad_e = EPAD - (e + N)
    src_all = jnp.concatenate(
        [edge_index[0], iota, jnp.zeros((npad_e,), jnp.int32)])
    pad_dst = DUMMY + jnp.arange(npad_e, dtype=jnp.int32) % (NPAD - N)
    dst_all = jnp.concatenate([edge_index[1], iota, pad_dst])
    src3 = src_all.reshape(NW, CPT, K)
    dst3 = dst_all.reshape(NW, CPT, K)
    ones16 = jnp.ones((K, 16), jnp.float32)
    zeros16 = jnp.zeros((NPAD, 16), jnp.float32)
    zeros_d = jnp.zeros((NPAD, D), jnp.float32)

    degacc = _degree_histogram(dst3, ones16, zeros16)

    h = pl.pallas_call(
        _matmul_body,
        out_shape=jax.ShapeDtypeStruct((N, D), jnp.float32),
    )(x, W1)

    h2 = pl.pallas_call(
        _scale_body,
        out_shape=jax.ShapeDtypeStruct((N, D), jnp.float32),
    )(h, degacc)

    seg = _segment_scatter(h2, src3, dst3, zeros_d)

    out = pl.pallas_call(
        _combine_body,
        out_shape=jax.ShapeDtypeStruct((1, FOUT), jnp.float32),
    )(seg, degacc, b1.reshape(1, D), W2, b2.reshape(1, FOUT))
    return out


# speedup vs baseline: 1.9026x; 1.0060x over previous
import functools

import jax
import jax.numpy as jnp
from jax import lax
from jax.experimental import pallas as pl
from jax.experimental.pallas import tpu as pltpu
from jax.experimental.pallas import tpu_sc as plsc

N = 10000
D = 128
FOUT = 2
NPAD = 10112
DUMMY = N
NC, NS = 2, 16
NW = NC * NS
K = 128
CPT = 81
EPAD = NW * CPT * K
RPT = NPAD // NS

_mesh = plsc.VectorSubcoreMesh(core_axis_name="c", subcore_axis_name="s",
                               num_cores=NC, num_subcores=NS)


@functools.partial(
    pl.kernel,
    mesh=_mesh,
    out_type=jax.ShapeDtypeStruct((NC, NPAD, 16), jnp.float32),
    scratch_types=[
        pltpu.VMEM((CPT, K), jnp.int32),
        pltpu.VMEM((K, 16), jnp.float32),
        pltpu.VMEM_SHARED((NPAD, 16), jnp.float32),
    ],
)
def _degree_histogram(dst_hbm, ones_hbm, zeros_hbm, out_hbm,
                      idx_v, ones_v, acc_sh):
    c = lax.axis_index("c")
    s = lax.axis_index("s")
    w = c * NS + s
    pltpu.sync_copy(dst_hbm.at[w], idx_v)
    pltpu.sync_copy(ones_hbm, ones_v)
    r0 = s * RPT
    pltpu.sync_copy(zeros_hbm.at[pl.ds(r0, RPT)], acc_sh.at[pl.ds(r0, RPT)])
    plsc.subcore_barrier()

    @pl.loop(0, CPT)
    def _(j):
        pltpu.sync_copy(ones_v, acc_sh.at[idx_v.at[j]], add=True)

    plsc.subcore_barrier()
    pltpu.sync_copy(acc_sh.at[pl.ds(r0, RPT)], out_hbm.at[c, pl.ds(r0, RPT)])


@functools.partial(
    pl.kernel,
    mesh=_mesh,
    out_type=jax.ShapeDtypeStruct((NC, NPAD, D), jnp.float32),
    scratch_types=[
        pltpu.VMEM((CPT, K), jnp.int32),
        pltpu.VMEM((CPT, K), jnp.int32),
        pltpu.VMEM((K, D), jnp.float32),
        pltpu.VMEM_SHARED((NPAD, D), jnp.float32),
        pltpu.SemaphoreType.DMA,
    ],
)
def _segment_scatter(h2_hbm, src_hbm, dst_hbm, zeros_hbm, out_hbm,
                     src_v, dst_v, rows_v, acc_sh, sem):
    c = lax.axis_index("c")
    s = lax.axis_index("s")
    w = s * NC + c
    pltpu.sync_copy(src_hbm.at[w], src_v)
    pltpu.sync_copy(dst_hbm.at[w], dst_v)
    r0 = s * RPT
    pltpu.sync_copy(zeros_hbm.at[pl.ds(r0, RPT)], acc_sh.at[pl.ds(r0, RPT)])
    plsc.subcore_barrier()

    @pl.loop(0, CPT)
    def _(j):
        pltpu.async_copy(h2_hbm.at[src_v.at[j]], rows_v, sem).wait()
        pltpu.sync_copy(rows_v, acc_sh.at[dst_v.at[j]], add=True)

    plsc.subcore_barrier()
    pltpu.sync_copy(acc_sh.at[pl.ds(r0, RPT)], out_hbm.at[c, pl.ds(r0, RPT)])


def _matmul_body(x_ref, w1_ref, h_ref):
    h_ref[...] = jnp.dot(x_ref[...], w1_ref[...],
                         preferred_element_type=jnp.float32)


def _scale_body(h_ref, degacc_ref, h2_ref):
    deg = degacc_ref[0, :, 0:1] + degacc_ref[1, :, 0:1]
    dinv = lax.rsqrt(deg[:N])
    h2_ref[...] = h_ref[...] * dinv


def _combine_body(s_ref, degacc_ref, b1_ref, w2_ref, b2_ref, out_ref):
    deg = degacc_ref[0, :, 0:1] + degacc_ref[1, :, 0:1]
    dinv = lax.rsqrt(deg[:N])
    srows = s_ref[0, :N, :] + s_ref[1, :N, :]
    rows = jnp.maximum(srows * dinv + b1_ref[...], 0.0)
    g = jnp.sum(rows, axis=0, keepdims=True) * (1.0 / N)
    logits = jnp.dot(g, w2_ref[...], preferred_element_type=jnp.float32)
    logits = logits + b2_ref[...]
    m = jnp.max(logits, axis=1, keepdims=True)
    e = jnp.exp(logits - m)
    out_ref[...] = e / jnp.sum(e, axis=1, keepdims=True)


def kernel(x, edge_index, W1, b1, W2, b2):
    e = edge_index.shape[1]
    iota = jnp.arange(N, dtype=jnp.int32)
    npad_e = EPAD - (e + N)
    src_all = jnp.concatenate(
        [edge_index[0], iota, jnp.zeros((npad_e,), jnp.int32)])
    pad_dst = DUMMY + jnp.arange(npad_e, dtype=jnp.int32) % (NPAD - N)
    dst_all = jnp.concatenate([edge_index[1], iota, pad_dst])
    src3 = src_all.reshape(NW, CPT, K)
    dst3 = dst_all.reshape(NW, CPT, K)
    ones16 = jnp.ones((K, 16), jnp.float32)
    zeros16 = jnp.zeros((NPAD, 16), jnp.float32)
    zeros_d = jnp.zeros((NPAD, D), jnp.float32)

    degacc = _degree_histogram(dst3, ones16, zeros16)

    h = pl.pallas_call(
        _matmul_body,
        out_shape=jax.ShapeDtypeStruct((N, D), jnp.float32),
    )(x, W1)

    h2 = pl.pallas_call(
        _scale_body,
        out_shape=jax.ShapeDtypeStruct((N, D), jnp.float32),
    )(h, degacc)

    seg = _segment_scatter(h2, src3, dst3, zeros_d)

    out = pl.pallas_call(
        _combine_body,
        out_shape=jax.ShapeDtypeStruct((1, FOUT), jnp.float32),
    )(seg, degacc, b1.reshape(1, D), W2, b2.reshape(1, FOUT))
    return out
